# baseline (speedup 1.0000x reference)
import functools

import jax
import jax.numpy as jnp
from jax import lax
from jax.experimental import pallas as pl
from jax.experimental.pallas import tpu as pltpu
from jax.experimental.pallas import tpu_sc as plsc

SPLITS = (512, 1536)


def _sc_gather_chunk(ids_chunk, embed_table, rows, H):
    info = plsc.get_sparse_core_info()
    NC, NS = info.num_cores, info.num_subcores
    NW = NC * NS
    b_per_w = rows // NW
    CH = 16
    NCH = b_per_w // CH

    mesh = plsc.VectorSubcoreMesh(core_axis_name="c", subcore_axis_name="s")

    @functools.partial(
        pl.kernel,
        mesh=mesh,
        out_type=jax.ShapeDtypeStruct((rows, H), jnp.float32),
        scratch_types=[
            pltpu.VMEM((max(NCH, 2), CH), jnp.int32),
            pltpu.VMEM((CH, H), jnp.float32),
            pltpu.VMEM((CH, H), jnp.float32),
            pltpu.SemaphoreType.DMA,
            pltpu.SemaphoreType.DMA,
        ],
    )
    def gather_kernel(idx_hbm, table_hbm, out_hbm, idx_v, buf0, buf1, sem0, sem1):
        wid = lax.axis_index("s") * NC + lax.axis_index("c")
        base = wid * b_per_w
        pltpu.sync_copy(idx_hbm.at[wid], idx_v)
        bufs = (buf0, buf1)
        sems = (sem0, sem1)
        cps = [None] * NCH
        cps[0] = pltpu.async_copy(table_hbm.at[idx_v.at[0]], buf0, sem0)
        for c in range(NCH):
            nxt = c + 1
            if nxt < NCH:
                cps[nxt] = pltpu.async_copy(
                    table_hbm.at[idx_v.at[nxt]], bufs[nxt % 2], sems[nxt % 2]
                )
            cps[c].wait()
            pltpu.sync_copy(bufs[c % 2], out_hbm.at[pl.ds(base + c * CH, CH)])

    ids3 = jnp.pad(ids_chunk.reshape(NW, NCH, CH), ((0, 0), (0, max(NCH, 2) - NCH), (0, 0)))
    return gather_kernel(ids3, embed_table)


def _tc_matmul_chunk(y_buf, x_c, W, b2, row0, rows, S, H):
    BS = 256
    nblk = rows // BS
    blk0 = row0 // BS

    def mm_body(*refs):
        x_ref, w_ref, b_ref, y_ref = refs[-4:]
        y_ref[...] = (
            lax.dot_general(
                x_ref[...],
                w_ref[...],
                (((1,), (1,)), ((), ())),
                preferred_element_type=jnp.float32,
            )
            + b_ref[...]
        )

    data_specs = [
        pl.BlockSpec((BS, H), lambda i: (i, 0)),
        pl.BlockSpec((H, H), lambda i: (0, 0)),
        pl.BlockSpec((1, H), lambda i: (0, 0)),
    ]
    if y_buf is None:
        in_specs = data_specs
        args = (x_c, W, b2)
        aliases = {}
    else:
        in_specs = [pl.BlockSpec(memory_space=pl.ANY)] + data_specs
        args = (y_buf, x_c, W, b2)
        aliases = {0: 0}

    return pl.pallas_call(
        mm_body,
        grid=(nblk,),
        in_specs=in_specs,
        out_specs=pl.BlockSpec((BS, H), lambda i: (blk0 + i, 0)),
        out_shape=jax.ShapeDtypeStruct((S, H), jnp.float32),
        input_output_aliases=aliases,
    )(*args)


def kernel(input_ids, embed_table, W, b):
    B, S = input_ids.shape
    V, H = embed_table.shape
    SR = B * S
    ids_flat = input_ids.reshape(SR).astype(jnp.int32)
    b2 = b.reshape(1, H)

    xs = []
    row0 = 0
    for rows in SPLITS:
        xs.append(_sc_gather_chunk(ids_flat[row0 : row0 + rows], embed_table, rows, H))
        row0 += rows
    y = None
    row0 = 0
    for c, rows in enumerate(SPLITS):
        y = _tc_matmul_chunk(y, xs[c], W, b2, row0, rows, SR, H)
        row0 += rows
    return y.reshape(B, S, H)


# speedup vs baseline: 1.0602x; 1.0602x over previous
import functools

import jax
import jax.numpy as jnp
from jax import lax
from jax.experimental import pallas as pl
from jax.experimental.pallas import tpu as pltpu
from jax.experimental.pallas import tpu_sc as plsc


def _sc_gather(ids, embed_table, S, H):
    info = plsc.get_sparse_core_info()
    NC, NS = info.num_cores, info.num_subcores
    NW = NC * NS
    b_per_w = S // NW
    CH = 16
    NCH = b_per_w // CH

    mesh = plsc.VectorSubcoreMesh(core_axis_name="c", subcore_axis_name="s")

    @functools.partial(
        pl.kernel,
        mesh=mesh,
        out_type=jax.ShapeDtypeStruct((S, H), jnp.float32),
        scratch_types=[
            pltpu.VMEM((NCH * CH,), jnp.int32),
            pltpu.VMEM((CH, H), jnp.float32),
            pltpu.VMEM((CH, H), jnp.float32),
            pltpu.SemaphoreType.DMA,
            pltpu.SemaphoreType.DMA,
        ],
    )
    def gather_kernel(idx_hbm, table_hbm, out_hbm, idx_v, buf0, buf1, sem0, sem1):
        wid = lax.axis_index("s") * NC + lax.axis_index("c")
        base = wid * b_per_w
        pltpu.sync_copy(idx_hbm.at[pl.ds(base, b_per_w)], idx_v)
        bufs = (buf0, buf1)
        sems = (sem0, sem1)
        cps = [None] * NCH
        cps[0] = pltpu.async_copy(
            table_hbm.at[idx_v.at[pl.ds(0, CH)]], buf0, sem0
        )
        for c in range(NCH):
            nxt = c + 1
            if nxt < NCH:
                cps[nxt] = pltpu.async_copy(
                    table_hbm.at[idx_v.at[pl.ds(nxt * CH, CH)]],
                    bufs[nxt % 2],
                    sems[nxt % 2],
                )
            cps[c].wait()
            pltpu.sync_copy(bufs[c % 2], out_hbm.at[pl.ds(base + c * CH, CH)])

    return gather_kernel(ids, embed_table)


def _tc_matmul(x, W, b2, S, H):
    BS = 512

    def mm_body(x_ref, w_ref, b_ref, y_ref):
        y_ref[...] = (
            lax.dot_general(
                x_ref[...],
                w_ref[...],
                (((1,), (1,)), ((), ())),
                preferred_element_type=jnp.float32,
            )
            + b_ref[...]
        )

    return pl.pallas_call(
        mm_body,
        grid=(S // BS,),
        in_specs=[
            pl.BlockSpec((BS, H), lambda i: (i, 0)),
            pl.BlockSpec((H, H), lambda i: (0, 0)),
            pl.BlockSpec((1, H), lambda i: (0, 0)),
        ],
        out_specs=pl.BlockSpec((BS, H), lambda i: (i, 0)),
        out_shape=jax.ShapeDtypeStruct((S, H), jnp.float32),
    )(x, W, b2)


def kernel(input_ids, embed_table, W, b):
    B, S = input_ids.shape
    V, H = embed_table.shape
    SR = B * S
    ids_flat = input_ids.reshape(SR).astype(jnp.int32)
    x = _sc_gather(ids_flat, embed_table, SR, H)
    y = _tc_matmul(x, W, b.reshape(1, H), SR, H)
    return y.reshape(B, S, H)
